# baseline (speedup 1.0000x reference)
import functools

import jax
import jax.numpy as jnp
from jax.experimental import pallas as pl
from jax.experimental.pallas import tpu as pltpu

N = 10000
E = 160000
D = 256
P = 8192

BN = 400
BP = 512


def _combine_body(h_ref, a_ref, ws_ref, wn_ref, b_ref, o_ref, *, relu):
    acc = jnp.dot(h_ref[...], ws_ref[...], preferred_element_type=jnp.float32)
    acc = acc + jnp.dot(a_ref[...], wn_ref[...], preferred_element_type=jnp.float32)
    acc = acc + b_ref[...]
    if relu:
        acc = jnp.maximum(acc, 0.0)
    o_ref[...] = acc


def _combine(h, aggm, W_self, W_neigh, b, relu):
    return pl.pallas_call(
        functools.partial(_combine_body, relu=relu),
        grid=(N // BN,),
        in_specs=[
            pl.BlockSpec((BN, D), lambda i: (i, 0)),
            pl.BlockSpec((BN, D), lambda i: (i, 0)),
            pl.BlockSpec((D, D), lambda i: (0, 0)),
            pl.BlockSpec((D, D), lambda i: (0, 0)),
            pl.BlockSpec((1, D), lambda i: (0, 0)),
        ],
        out_specs=pl.BlockSpec((BN, D), lambda i: (i, 0)),
        out_shape=jax.ShapeDtypeStruct((N, D), jnp.float32),
    )(h, aggm, W_self, W_neigh, b.reshape(1, D))


def _decoder_body(zp_ref, zn_ref, w1_ref, b1_ref, w2_ref, b2_ref, w3_ref,
                  b3_ref, op_ref, on_ref):
    for z_ref, o_ref in ((zp_ref, op_ref), (zn_ref, on_ref)):
        z = jnp.dot(z_ref[...], w1_ref[...], preferred_element_type=jnp.float32)
        z = jnp.maximum(z + b1_ref[...], 0.0)
        z = jnp.dot(z, w2_ref[...], preferred_element_type=jnp.float32)
        z = jnp.maximum(z + b2_ref[...], 0.0)
        o_ref[...] = jnp.sum(z * w3_ref[...], axis=1, keepdims=True) + b3_ref[...]


def _decoder(zp, zn, dW1, db1, dW2, db2, dW3, db3):
    return pl.pallas_call(
        _decoder_body,
        grid=(P // BP,),
        in_specs=[
            pl.BlockSpec((BP, D), lambda i: (i, 0)),
            pl.BlockSpec((BP, D), lambda i: (i, 0)),
            pl.BlockSpec((D, D), lambda i: (0, 0)),
            pl.BlockSpec((1, D), lambda i: (0, 0)),
            pl.BlockSpec((D, D), lambda i: (0, 0)),
            pl.BlockSpec((1, D), lambda i: (0, 0)),
            pl.BlockSpec((1, D), lambda i: (0, 0)),
            pl.BlockSpec((1, 1), lambda i: (0, 0)),
        ],
        out_specs=[
            pl.BlockSpec((BP, 1), lambda i: (i, 0)),
            pl.BlockSpec((BP, 1), lambda i: (i, 0)),
        ],
        out_shape=[
            jax.ShapeDtypeStruct((P, 1), jnp.float32),
            jax.ShapeDtypeStruct((P, 1), jnp.float32),
        ],
    )(zp, zn, dW1, db1.reshape(1, D), dW2, db2.reshape(1, D),
      dW3.reshape(1, D), db3.reshape(1, 1))


def _agg_mean(h, edge_index):
    src = edge_index[0]
    dst = edge_index[1]
    agg = jax.ops.segment_sum(h[src], dst, num_segments=N)
    deg = jax.ops.segment_sum(jnp.ones((edge_index.shape[1],), h.dtype), dst,
                              num_segments=N)
    return agg / jnp.maximum(deg, 1.0)[:, None]


def kernel(x, edge_index1, edge_index2, pos_src, pos_dst, neg_src, neg_dst,
           emb, W_self1, W_neigh1, b1, W_self2, W_neigh2, b2,
           dW1, db1, dW2, db2, dW3, db3):
    h = emb[x]
    aggm1 = _agg_mean(h, edge_index1)
    h = _combine(h, aggm1, W_self1, W_neigh1, b1, relu=True)
    aggm2 = _agg_mean(h, edge_index2)
    h = _combine(h, aggm2, W_self2, W_neigh2, b2, relu=False)
    zp = h[pos_src] * h[pos_dst]
    zn = h[neg_src] * h[neg_dst]
    return tuple(_decoder(zp, zn, dW1, db1, dW2, db2, dW3, db3))


# speedup vs baseline: 3.8010x; 3.8010x over previous
import dataclasses
import functools

import jax
import jax.numpy as jnp
from jax import lax
from jax.experimental import pallas as pl
from jax.experimental.pallas import tpu as pltpu
from jax.experimental.pallas import tpu_sc as plsc

N = 10000
E = 160000
D = 256
H = 128
P = 8192
ER = E // 128
ERP = 1256
PR = P // 128
NCH = 78

BN = 400
BP = 512

_MESH = plsc.VectorSubcoreMesh(core_axis_name="c", subcore_axis_name="s")

_SC_PARAMS = pltpu.CompilerParams()
if "needs_layout_passes" in pltpu.CompilerParams.__dataclass_fields__:
    _SC_PARAMS = dataclasses.replace(_SC_PARAMS, needs_layout_passes=False)


def _f32(shape):
    return jax.ShapeDtypeStruct(shape, jnp.float32)


def _i32(shape):
    return jax.ShapeDtypeStruct(shape, jnp.int32)


@functools.partial(
    pl.kernel,
    mesh=_MESH,
    out_type=[_f32((N, D))],
    scratch_types=[
        pltpu.VMEM((1, 128), jnp.int32),
        pltpu.VMEM((128, D), jnp.float32),
    ],
    compiler_params=_SC_PARAMS,
)
def _sc_prepare(xp2_hbm, emb_hbm, h0_hbm, idx_v, rows_v):
    w = lax.axis_index("s") * 2 + lax.axis_index("c")

    @pl.loop(w, NCH, step=32)
    def _h0_chunk(cid):
        pltpu.sync_copy(xp2_hbm.at[pl.ds(cid, 1)], idx_v.at[pl.ds(0, 1)])
        pltpu.sync_copy(emb_hbm.at[idx_v.at[0]], rows_v)
        pltpu.sync_copy(rows_v, h0_hbm.at[pl.ds(cid * 128, 128)])

    @pl.when(w == 14)
    def _h0_tail():
        pltpu.sync_copy(xp2_hbm.at[pl.ds(NCH, 1)], idx_v.at[pl.ds(0, 1)])
        pltpu.sync_copy(emb_hbm.at[idx_v.at[0]], rows_v)
        pltpu.sync_copy(rows_v.at[pl.ds(0, 16)], h0_hbm.at[pl.ds(NCH * 128, 16)])


@functools.partial(
    pl.kernel,
    mesh=_MESH,
    out_type=[_f32((2 * N, H))],
    scratch_types=[
        pltpu.VMEM((8, 128), jnp.int32),
        pltpu.VMEM((128, H), jnp.float32),
        pltpu.VMEM((208, H), jnp.float32),
        pltpu.VMEM_SHARED((N, H), jnp.float32),
    ],
)
def _sc_degree(dst2_hbm, deg_hbm, idx_d, ones_v, zbuf16, dacc):
    cid = lax.axis_index("c")
    s = lax.axis_index("s")
    zero16 = jnp.zeros((16,), jnp.float32)
    one16 = jnp.ones((16,), jnp.float32)

    @pl.loop(0, 208)
    def _zf(r):
        for k in range(8):
            zbuf16[r, pl.ds(k * 16, 16)] = zero16

    @pl.loop(0, 128)
    def _of(r):
        for k in range(8):
            ones_v[r, pl.ds(k * 16, 16)] = one16

    for k in range(3):
        pltpu.sync_copy(zbuf16, dacc.at[pl.ds(s * 624 + k * 208, 208)])

    @pl.when(s == 0)
    def _ztail():
        pltpu.sync_copy(zbuf16.at[pl.ds(0, 16)], dacc.at[pl.ds(9984, 16)])

    plsc.subcore_barrier()

    @pl.loop(s, 157, step=16)
    def _edges(m):
        def _rows(nrows):
            pltpu.sync_copy(dst2_hbm.at[pl.ds(cid * ERP + m * 8, nrows)],
                            idx_d.at[pl.ds(0, nrows)])
            for j in range(nrows):
                pltpu.sync_copy(ones_v, dacc.at[idx_d.at[j]], add=True)

        @pl.when(m < 156)
        def _full():
            _rows(8)

        @pl.when(m == 156)
        def _tail():
            _rows(2)

    plsc.subcore_barrier()

    for k in range(3):
        rs = pl.ds(s * 624 + k * 208, 208)
        os = pl.ds(cid * N + s * 624 + k * 208, 208)
        pltpu.sync_copy(dacc.at[rs], deg_hbm.at[os])

    @pl.when(s == 0)
    def _otail():
        pltpu.sync_copy(dacc.at[pl.ds(9984, 16)],
                        deg_hbm.at[pl.ds(cid * N + 9984, 16)])


@functools.partial(
    pl.kernel,
    mesh=_MESH,
    out_type=[_f32((2 * N, H))],
    scratch_types=[
        pltpu.VMEM((8, 128), jnp.int32),
        pltpu.VMEM((8, 128), jnp.int32),
        pltpu.VMEM((1, 128), jnp.int32),
        pltpu.VMEM((128, H), jnp.float32),
        pltpu.VMEM((208, H), jnp.float32),
        pltpu.VMEM_SHARED((N, H), jnp.float32),
    ],
)
def _sc_aggregate(hr_hbm, srcr_hbm, dstr_hbm, agg_hbm,
                  idx_s, idx_d, gidx, rows_v, zbuf, acc):
    cid = lax.axis_index("c")
    s = lax.axis_index("s")
    zero16 = jnp.zeros((16,), jnp.float32)

    @pl.loop(0, 208)
    def _zfill(r):
        for k in range(8):
            zbuf[r, pl.ds(k * 16, 16)] = zero16

    for k in range(3):
        pltpu.sync_copy(zbuf, acc.at[pl.ds(s * 624 + k * 208, 208)])

    @pl.when(s == 0)
    def _ztail():
        pltpu.sync_copy(zbuf.at[pl.ds(0, 16)], acc.at[pl.ds(9984, 16)])

    plsc.subcore_barrier()

    @pl.loop(s, 157, step=16)
    def _edges(m):
        def _rows(nrows):
            pltpu.sync_copy(srcr_hbm.at[pl.ds(m * 8, nrows)],
                            idx_s.at[pl.ds(0, nrows)])
            pltpu.sync_copy(dstr_hbm.at[pl.ds(m * 8, nrows)],
                            idx_d.at[pl.ds(0, nrows)])
            for j in range(nrows):
                for k in range(8):
                    sl = pl.ds(k * 16, 16)
                    gidx[0, sl] = idx_s[j, sl] * 2 + cid
                pltpu.sync_copy(hr_hbm.at[gidx.at[0]], rows_v)
                pltpu.sync_copy(rows_v, acc.at[idx_d.at[j]], add=True)

        @pl.when(m < 156)
        def _full():
            _rows(8)

        @pl.when(m == 156)
        def _tail():
            _rows(2)

    plsc.subcore_barrier()

    for k in range(3):
        rs = pl.ds(s * 624 + k * 208, 208)
        os = pl.ds(cid * N + s * 624 + k * 208, 208)
        pltpu.sync_copy(acc.at[rs], agg_hbm.at[os])

    @pl.when(s == 0)
    def _otail():
        pltpu.sync_copy(acc.at[pl.ds(9984, 16)],
                        agg_hbm.at[pl.ds(cid * N + 9984, 16)])


@functools.partial(
    pl.kernel,
    mesh=_MESH,
    out_type=[_f32((4 * P, H))],
    scratch_types=[
        pltpu.VMEM((8, 128), jnp.int32),
        pltpu.VMEM((8, 128), jnp.int32),
        pltpu.VMEM((1, 128), jnp.int32),
        pltpu.VMEM((1, 128), jnp.int32),
        pltpu.VMEM((128, H), jnp.float32),
        pltpu.VMEM((128, H), jnp.float32),
    ],
)
def _sc_pairs(hr_hbm, pairs_hbm, z_hbm, ia, ib, ga, gb, ra, rb):
    cid = lax.axis_index("c")
    s = lax.axis_index("s")

    side = s // 8
    m = s % 8
    pltpu.sync_copy(pairs_hbm.at[pl.ds(side * 2 * PR + m * 8, 8)], ia)
    pltpu.sync_copy(pairs_hbm.at[pl.ds(side * 2 * PR + PR + m * 8, 8)], ib)
    for j in range(8):
        for k in range(8):
            sl = pl.ds(k * 16, 16)
            ga[0, sl] = ia[j, sl] * 2 + cid
            gb[0, sl] = ib[j, sl] * 2 + cid
        pltpu.sync_copy(hr_hbm.at[ga.at[0]], ra)
        pltpu.sync_copy(hr_hbm.at[gb.at[0]], rb)

        @pl.loop(0, 128)
        def _mul(i):
            for k in range(8):
                sl = pl.ds(k * 16, 16)
                ra[i, sl] = ra[i, sl] * rb[i, sl]

        pltpu.sync_copy(
            ra, z_hbm.at[pl.ds(side * 2 * P + cid * P + (m * 8 + j) * 128, 128)])


def _combine_body(h_ref, alo_ref, ahi_ref, deg_ref, ws_ref, wn_ref, b_ref,
                  o_ref, *, relu):
    inv = 1.0 / jnp.maximum(deg_ref[...][:, 0:1], 1.0)
    acc = jnp.dot(h_ref[...], ws_ref[...], preferred_element_type=jnp.float32)
    acc = acc + jnp.dot(alo_ref[...] * inv, wn_ref[...][:H, :],
                        preferred_element_type=jnp.float32)
    acc = acc + jnp.dot(ahi_ref[...] * inv, wn_ref[...][H:, :],
                        preferred_element_type=jnp.float32)
    acc = acc + b_ref[...]
    if relu:
        acc = jnp.maximum(acc, 0.0)
    o_ref[...] = acc


def _combine(h, agg_lo, agg_hi, deg, W_self, W_neigh, b, relu):
    return pl.pallas_call(
        functools.partial(_combine_body, relu=relu),
        grid=(N // BN,),
        in_specs=[
            pl.BlockSpec((BN, D), lambda i: (i, 0)),
            pl.BlockSpec((BN, H), lambda i: (i, 0)),
            pl.BlockSpec((BN, H), lambda i: (i, 0)),
            pl.BlockSpec((BN, H), lambda i: (i, 0)),
            pl.BlockSpec((D, D), lambda i: (0, 0)),
            pl.BlockSpec((D, D), lambda i: (0, 0)),
            pl.BlockSpec((1, D), lambda i: (0, 0)),
        ],
        out_specs=pl.BlockSpec((BN, D), lambda i: (i, 0)),
        out_shape=_f32((N, D)),
    )(h, agg_lo, agg_hi, deg, W_self, W_neigh, b.reshape(1, D))


def _decoder_body(zpl_ref, zph_ref, znl_ref, znh_ref, w1_ref, b1_ref, w2_ref,
                  b2_ref, w3_ref, b3_ref, op_ref, on_ref):
    for zl_ref, zh_ref, o_ref in ((zpl_ref, zph_ref, op_ref),
                                  (znl_ref, znh_ref, on_ref)):
        z = jnp.dot(zl_ref[...], w1_ref[...][:H, :],
                    preferred_element_type=jnp.float32)
        z = z + jnp.dot(zh_ref[...], w1_ref[...][H:, :],
                        preferred_element_type=jnp.float32)
        z = jnp.maximum(z + b1_ref[...], 0.0)
        z = jnp.dot(z, w2_ref[...], preferred_element_type=jnp.float32)
        z = jnp.maximum(z + b2_ref[...], 0.0)
        o_ref[...] = jnp.sum(z * w3_ref[...], axis=1, keepdims=True) + b3_ref[...]


def _decoder(zp_lo, zp_hi, zn_lo, zn_hi, dW1, db1, dW2, db2, dW3, db3):
    return pl.pallas_call(
        _decoder_body,
        grid=(P // BP,),
        in_specs=[
            pl.BlockSpec((BP, H), lambda i: (i, 0)),
            pl.BlockSpec((BP, H), lambda i: (i, 0)),
            pl.BlockSpec((BP, H), lambda i: (i, 0)),
            pl.BlockSpec((BP, H), lambda i: (i, 0)),
            pl.BlockSpec((D, D), lambda i: (0, 0)),
            pl.BlockSpec((1, D), lambda i: (0, 0)),
            pl.BlockSpec((D, D), lambda i: (0, 0)),
            pl.BlockSpec((1, D), lambda i: (0, 0)),
            pl.BlockSpec((1, D), lambda i: (0, 0)),
            pl.BlockSpec((1, 1), lambda i: (0, 0)),
        ],
        out_specs=[
            pl.BlockSpec((BP, 1), lambda i: (i, 0)),
            pl.BlockSpec((BP, 1), lambda i: (i, 0)),
        ],
        out_shape=[_f32((P, 1)), _f32((P, 1))],
    )(zp_lo, zp_hi, zn_lo, zn_hi, dW1, db1.reshape(1, D), dW2,
      db2.reshape(1, D), dW3.reshape(1, D), db3.reshape(1, 1))


def kernel(x, edge_index1, edge_index2, pos_src, pos_dst, neg_src, neg_dst,
           emb, W_self1, W_neigh1, b1, W_self2, W_neigh2, b2,
           dW1, db1, dW2, db2, dW3, db3):
    i32 = jnp.int32
    x = x.astype(i32)
    xp2 = jnp.concatenate([x, jnp.zeros((10240 - N,), i32)]).reshape(80, 128)
    src1r = edge_index1[0].astype(i32).reshape(ER, 128)
    dst1r = edge_index1[1].astype(i32).reshape(ER, 128)
    src2r = edge_index2[0].astype(i32).reshape(ER, 128)
    dst2r = edge_index2[1].astype(i32).reshape(ER, 128)
    zpad = jnp.zeros((ERP - ER, 128), i32)
    dst_both = jnp.concatenate([dst1r, zpad, dst2r, zpad], axis=0)
    pairs = jnp.concatenate(
        [pos_src.astype(i32).reshape(PR, 128),
         pos_dst.astype(i32).reshape(PR, 128),
         neg_src.astype(i32).reshape(PR, 128),
         neg_dst.astype(i32).reshape(PR, 128)], axis=0)

    (h0,) = _sc_prepare(xp2, emb)
    (deg_both,) = _sc_degree(dst_both)
    (agg1,) = _sc_aggregate(h0.reshape(2 * N, H), src1r, dst1r)
    h1 = _combine(h0, agg1[:N], agg1[N:], deg_both[:N],
                  W_self1, W_neigh1, b1, relu=True)

    (agg2,) = _sc_aggregate(h1.reshape(2 * N, H), src2r, dst2r)
    h2 = _combine(h1, agg2[:N], agg2[N:], deg_both[N:],
                  W_self2, W_neigh2, b2, relu=False)

    (z,) = _sc_pairs(h2.reshape(2 * N, H), pairs)
    return tuple(_decoder(z[0:P], z[P:2 * P], z[2 * P:3 * P], z[3 * P:],
                          dW1, db1, dW2, db2, dW3, db3))


# speedup vs baseline: 4.3600x; 1.1471x over previous
import dataclasses
import functools

import jax
import jax.numpy as jnp
from jax import lax
from jax.experimental import pallas as pl
from jax.experimental.pallas import tpu as pltpu
from jax.experimental.pallas import tpu_sc as plsc

N = 10000
E = 160000
D = 256
H = 128
P = 8192
ER = E // 128
ERP = 1256
PR = P // 128
NCH = 78

BN = 400
BP = 512

_MESH = plsc.VectorSubcoreMesh(core_axis_name="c", subcore_axis_name="s")

_SC_PARAMS = pltpu.CompilerParams()
if "needs_layout_passes" in pltpu.CompilerParams.__dataclass_fields__:
    _SC_PARAMS = dataclasses.replace(_SC_PARAMS, needs_layout_passes=False)


def _f32(shape):
    return jax.ShapeDtypeStruct(shape, jnp.float32)


def _i32(shape):
    return jax.ShapeDtypeStruct(shape, jnp.int32)


@functools.partial(
    pl.kernel,
    mesh=_MESH,
    out_type=[_f32((N, D))],
    scratch_types=[
        pltpu.VMEM((1, 128), jnp.int32),
        pltpu.VMEM((128, D), jnp.float32),
    ],
    compiler_params=_SC_PARAMS,
)
def _sc_prepare(xp2_hbm, emb_hbm, h0_hbm, idx_v, rows_v):
    w = lax.axis_index("s") * 2 + lax.axis_index("c")

    @pl.loop(w, NCH, step=32)
    def _h0_chunk(cid):
        pltpu.sync_copy(xp2_hbm.at[pl.ds(cid, 1)], idx_v.at[pl.ds(0, 1)])
        pltpu.sync_copy(emb_hbm.at[idx_v.at[0]], rows_v)
        pltpu.sync_copy(rows_v, h0_hbm.at[pl.ds(cid * 128, 128)])

    @pl.when(w == 14)
    def _h0_tail():
        pltpu.sync_copy(xp2_hbm.at[pl.ds(NCH, 1)], idx_v.at[pl.ds(0, 1)])
        pltpu.sync_copy(emb_hbm.at[idx_v.at[0]], rows_v)
        pltpu.sync_copy(rows_v.at[pl.ds(0, 16)], h0_hbm.at[pl.ds(NCH * 128, 16)])


@functools.partial(
    pl.kernel,
    mesh=_MESH,
    out_type=[_f32((2 * N, H))],
    scratch_types=[
        pltpu.VMEM((8, 128), jnp.int32),
        pltpu.VMEM((128, H), jnp.float32),
        pltpu.VMEM((208, H), jnp.float32),
        pltpu.VMEM_SHARED((N, H), jnp.float32),
    ],
)
def _sc_degree(dst2_hbm, deg_hbm, idx_d, ones_v, zbuf16, dacc):
    cid = lax.axis_index("c")
    s = lax.axis_index("s")
    zero16 = jnp.zeros((16,), jnp.float32)
    one16 = jnp.ones((16,), jnp.float32)

    @pl.loop(0, 208)
    def _zf(r):
        for k in range(8):
            zbuf16[r, pl.ds(k * 16, 16)] = zero16

    @pl.loop(0, 128)
    def _of(r):
        for k in range(8):
            ones_v[r, pl.ds(k * 16, 16)] = one16

    for k in range(3):
        pltpu.sync_copy(zbuf16, dacc.at[pl.ds(s * 624 + k * 208, 208)])

    @pl.when(s == 0)
    def _ztail():
        pltpu.sync_copy(zbuf16.at[pl.ds(0, 16)], dacc.at[pl.ds(9984, 16)])

    plsc.subcore_barrier()

    @pl.loop(s, 157, step=16)
    def _edges(m):
        def _rows(nrows):
            pltpu.sync_copy(dst2_hbm.at[pl.ds(cid * ERP + m * 8, nrows)],
                            idx_d.at[pl.ds(0, nrows)])
            for j in range(nrows):
                pltpu.sync_copy(ones_v, dacc.at[idx_d.at[j]], add=True)

        @pl.when(m < 156)
        def _full():
            _rows(8)

        @pl.when(m == 156)
        def _tail():
            _rows(2)

    plsc.subcore_barrier()

    for k in range(3):
        rs = pl.ds(s * 624 + k * 208, 208)
        os = pl.ds(cid * N + s * 624 + k * 208, 208)
        pltpu.sync_copy(dacc.at[rs], deg_hbm.at[os])

    @pl.when(s == 0)
    def _otail():
        pltpu.sync_copy(dacc.at[pl.ds(9984, 16)],
                        deg_hbm.at[pl.ds(cid * N + 9984, 16)])


@functools.partial(
    pl.kernel,
    mesh=_MESH,
    out_type=[_f32((2 * N, H))],
    scratch_types=[
        pltpu.VMEM((8, 128), jnp.int32),
        pltpu.VMEM((8, 128), jnp.int32),
        pltpu.VMEM((8, 128), jnp.int32),
        pltpu.VMEM((128, H), jnp.float32),
        pltpu.VMEM((128, H), jnp.float32),
        pltpu.VMEM((104, H), jnp.float32),
        pltpu.VMEM_SHARED((N, H), jnp.float32),
        pltpu.SemaphoreType.DMA,
        pltpu.SemaphoreType.DMA,
    ],
)
def _sc_aggregate(hr_hbm, srcr_hbm, dstr_hbm, agg_hbm,
                  idx_s, idx_d, gidx, buf_a, buf_b, zbuf, acc,
                  sem_a, sem_b):
    cid = lax.axis_index("c")
    s = lax.axis_index("s")
    zero16 = jnp.zeros((16,), jnp.float32)

    @pl.loop(0, 104)
    def _zfill(r):
        for k in range(8):
            zbuf[r, pl.ds(k * 16, 16)] = zero16

    for k in range(6):
        pltpu.sync_copy(zbuf, acc.at[pl.ds(s * 624 + k * 104, 104)])

    @pl.when(s == 0)
    def _ztail():
        pltpu.sync_copy(zbuf.at[pl.ds(0, 16)], acc.at[pl.ds(9984, 16)])

    plsc.subcore_barrier()

    @pl.loop(s, 157, step=16)
    def _edges(m):
        def _rows(nrows):
            pltpu.sync_copy(srcr_hbm.at[pl.ds(m * 8, nrows)],
                            idx_s.at[pl.ds(0, nrows)])
            pltpu.sync_copy(dstr_hbm.at[pl.ds(m * 8, nrows)],
                            idx_d.at[pl.ds(0, nrows)])
            for j in range(nrows):
                for k in range(8):
                    sl = pl.ds(k * 16, 16)
                    gidx[j, sl] = idx_s[j, sl] * 2 + cid
            bufs = (buf_a, buf_b)
            sems = (sem_a, sem_b)
            pend = [None, None]
            pend[0] = pltpu.async_copy(hr_hbm.at[gidx.at[0]], bufs[0], sems[0])
            for j in range(nrows):
                pend[j % 2].wait()
                if j + 1 < nrows:
                    pend[(j + 1) % 2] = pltpu.async_copy(
                        hr_hbm.at[gidx.at[j + 1]], bufs[(j + 1) % 2],
                        sems[(j + 1) % 2])
                pltpu.sync_copy(bufs[j % 2], acc.at[idx_d.at[j]], add=True)

        @pl.when(m < 156)
        def _full():
            _rows(8)

        @pl.when(m == 156)
        def _tail():
            _rows(2)

    plsc.subcore_barrier()

    for k in range(3):
        rs = pl.ds(s * 624 + k * 208, 208)
        os = pl.ds(cid * N + s * 624 + k * 208, 208)
        pltpu.sync_copy(acc.at[rs], agg_hbm.at[os])

    @pl.when(s == 0)
    def _otail():
        pltpu.sync_copy(acc.at[pl.ds(9984, 16)],
                        agg_hbm.at[pl.ds(cid * N + 9984, 16)])


@functools.partial(
    pl.kernel,
    mesh=_MESH,
    out_type=[_f32((4 * P, H))],
    scratch_types=[
        pltpu.VMEM((8, 128), jnp.int32),
        pltpu.VMEM((8, 128), jnp.int32),
        pltpu.VMEM((1, 128), jnp.int32),
        pltpu.VMEM((1, 128), jnp.int32),
        pltpu.VMEM((128, H), jnp.float32),
        pltpu.VMEM((128, H), jnp.float32),
    ],
)
def _sc_pairs(hr_hbm, pairs_hbm, z_hbm, ia, ib, ga, gb, ra, rb):
    cid = lax.axis_index("c")
    s = lax.axis_index("s")

    side = s // 8
    m = s % 8
    pltpu.sync_copy(pairs_hbm.at[pl.ds(side * 2 * PR + m * 8, 8)], ia)
    pltpu.sync_copy(pairs_hbm.at[pl.ds(side * 2 * PR + PR + m * 8, 8)], ib)
    for j in range(8):
        for k in range(8):
            sl = pl.ds(k * 16, 16)
            ga[0, sl] = ia[j, sl] * 2 + cid
            gb[0, sl] = ib[j, sl] * 2 + cid
        pltpu.sync_copy(hr_hbm.at[ga.at[0]], ra)
        pltpu.sync_copy(hr_hbm.at[gb.at[0]], rb)

        @pl.loop(0, 128)
        def _mul(i):
            for k in range(8):
                sl = pl.ds(k * 16, 16)
                ra[i, sl] = ra[i, sl] * rb[i, sl]

        pltpu.sync_copy(
            ra, z_hbm.at[pl.ds(side * 2 * P + cid * P + (m * 8 + j) * 128, 128)])


def _combine_body(h_ref, alo_ref, ahi_ref, deg_ref, ws_ref, wn_ref, b_ref,
                  o_ref, *, relu):
    inv = 1.0 / jnp.maximum(deg_ref[...][:, 0:1], 1.0)
    acc = jnp.dot(h_ref[...], ws_ref[...], preferred_element_type=jnp.float32)
    acc = acc + jnp.dot(alo_ref[...] * inv, wn_ref[...][:H, :],
                        preferred_element_type=jnp.float32)
    acc = acc + jnp.dot(ahi_ref[...] * inv, wn_ref[...][H:, :],
                        preferred_element_type=jnp.float32)
    acc = acc + b_ref[...]
    if relu:
        acc = jnp.maximum(acc, 0.0)
    o_ref[...] = acc


def _combine(h, agg_lo, agg_hi, deg, W_self, W_neigh, b, relu):
    return pl.pallas_call(
        functools.partial(_combine_body, relu=relu),
        grid=(N // BN,),
        in_specs=[
            pl.BlockSpec((BN, D), lambda i: (i, 0)),
            pl.BlockSpec((BN, H), lambda i: (i, 0)),
            pl.BlockSpec((BN, H), lambda i: (i, 0)),
            pl.BlockSpec((BN, H), lambda i: (i, 0)),
            pl.BlockSpec((D, D), lambda i: (0, 0)),
            pl.BlockSpec((D, D), lambda i: (0, 0)),
            pl.BlockSpec((1, D), lambda i: (0, 0)),
        ],
        out_specs=pl.BlockSpec((BN, D), lambda i: (i, 0)),
        out_shape=_f32((N, D)),
    )(h, agg_lo, agg_hi, deg, W_self, W_neigh, b.reshape(1, D))


def _decoder_body(zpl_ref, zph_ref, znl_ref, znh_ref, w1_ref, b1_ref, w2_ref,
                  b2_ref, w3_ref, b3_ref, op_ref, on_ref):
    for zl_ref, zh_ref, o_ref in ((zpl_ref, zph_ref, op_ref),
                                  (znl_ref, znh_ref, on_ref)):
        z = jnp.dot(zl_ref[...], w1_ref[...][:H, :],
                    preferred_element_type=jnp.float32)
        z = z + jnp.dot(zh_ref[...], w1_ref[...][H:, :],
                        preferred_element_type=jnp.float32)
        z = jnp.maximum(z + b1_ref[...], 0.0)
        z = jnp.dot(z, w2_ref[...], preferred_element_type=jnp.float32)
        z = jnp.maximum(z + b2_ref[...], 0.0)
        o_ref[...] = jnp.sum(z * w3_ref[...], axis=1, keepdims=True) + b3_ref[...]


def _decoder(zp_lo, zp_hi, zn_lo, zn_hi, dW1, db1, dW2, db2, dW3, db3):
    return pl.pallas_call(
        _decoder_body,
        grid=(P // BP,),
        in_specs=[
            pl.BlockSpec((BP, H), lambda i: (i, 0)),
            pl.BlockSpec((BP, H), lambda i: (i, 0)),
            pl.BlockSpec((BP, H), lambda i: (i, 0)),
            pl.BlockSpec((BP, H), lambda i: (i, 0)),
            pl.BlockSpec((D, D), lambda i: (0, 0)),
            pl.BlockSpec((1, D), lambda i: (0, 0)),
            pl.BlockSpec((D, D), lambda i: (0, 0)),
            pl.BlockSpec((1, D), lambda i: (0, 0)),
            pl.BlockSpec((1, D), lambda i: (0, 0)),
            pl.BlockSpec((1, 1), lambda i: (0, 0)),
        ],
        out_specs=[
            pl.BlockSpec((BP, 1), lambda i: (i, 0)),
            pl.BlockSpec((BP, 1), lambda i: (i, 0)),
        ],
        out_shape=[_f32((P, 1)), _f32((P, 1))],
    )(zp_lo, zp_hi, zn_lo, zn_hi, dW1, db1.reshape(1, D), dW2,
      db2.reshape(1, D), dW3.reshape(1, D), db3.reshape(1, 1))


def kernel(x, edge_index1, edge_index2, pos_src, pos_dst, neg_src, neg_dst,
           emb, W_self1, W_neigh1, b1, W_self2, W_neigh2, b2,
           dW1, db1, dW2, db2, dW3, db3):
    i32 = jnp.int32
    x = x.astype(i32)
    xp2 = jnp.concatenate([x, jnp.zeros((10240 - N,), i32)]).reshape(80, 128)
    src1r = edge_index1[0].astype(i32).reshape(ER, 128)
    dst1r = edge_index1[1].astype(i32).reshape(ER, 128)
    src2r = edge_index2[0].astype(i32).reshape(ER, 128)
    dst2r = edge_index2[1].astype(i32).reshape(ER, 128)
    zpad = jnp.zeros((ERP - ER, 128), i32)
    dst_both = jnp.concatenate([dst1r, zpad, dst2r, zpad], axis=0)
    pairs = jnp.concatenate(
        [pos_src.astype(i32).reshape(PR, 128),
         pos_dst.astype(i32).reshape(PR, 128),
         neg_src.astype(i32).reshape(PR, 128),
         neg_dst.astype(i32).reshape(PR, 128)], axis=0)

    (h0,) = _sc_prepare(xp2, emb)
    (deg_both,) = _sc_degree(dst_both)
    (agg1,) = _sc_aggregate(h0.reshape(2 * N, H), src1r, dst1r)
    h1 = _combine(h0, agg1[:N], agg1[N:], deg_both[:N],
                  W_self1, W_neigh1, b1, relu=True)

    (agg2,) = _sc_aggregate(h1.reshape(2 * N, H), src2r, dst2r)
    h2 = _combine(h1, agg2[:N], agg2[N:], deg_both[N:],
                  W_self2, W_neigh2, b2, relu=False)

    (z,) = _sc_pairs(h2.reshape(2 * N, H), pairs)
    return tuple(_decoder(z[0:P], z[P:2 * P], z[2 * P:3 * P], z[3 * P:],
                          dW1, db1, dW2, db2, dW3, db3))


# speedup vs baseline: 4.3655x; 1.0013x over previous
import dataclasses
import functools

import jax
import jax.numpy as jnp
from jax import lax
from jax.experimental import pallas as pl
from jax.experimental.pallas import tpu as pltpu
from jax.experimental.pallas import tpu_sc as plsc

N = 10000
E = 160000
D = 256
H = 128
P = 8192
ER = E // 128
ERP = 1256
PR = P // 128
NCH = 78

BN = 400
BP = 512

_MESH = plsc.VectorSubcoreMesh(core_axis_name="c", subcore_axis_name="s")

_SC_PARAMS = pltpu.CompilerParams()
if "needs_layout_passes" in pltpu.CompilerParams.__dataclass_fields__:
    _SC_PARAMS = dataclasses.replace(_SC_PARAMS, needs_layout_passes=False)


def _f32(shape):
    return jax.ShapeDtypeStruct(shape, jnp.float32)


def _i32(shape):
    return jax.ShapeDtypeStruct(shape, jnp.int32)


@functools.partial(
    pl.kernel,
    mesh=_MESH,
    out_type=[_f32((N, D))],
    scratch_types=[
        pltpu.VMEM((1, 128), jnp.int32),
        pltpu.VMEM((128, D), jnp.float32),
    ],
    compiler_params=_SC_PARAMS,
)
def _sc_prepare(xp2_hbm, emb_hbm, h0_hbm, idx_v, rows_v):
    w = lax.axis_index("s") * 2 + lax.axis_index("c")

    @pl.loop(w, NCH, step=32)
    def _h0_chunk(cid):
        pltpu.sync_copy(xp2_hbm.at[pl.ds(cid, 1)], idx_v.at[pl.ds(0, 1)])
        pltpu.sync_copy(emb_hbm.at[idx_v.at[0]], rows_v)
        pltpu.sync_copy(rows_v, h0_hbm.at[pl.ds(cid * 128, 128)])

    @pl.when(w == 14)
    def _h0_tail():
        pltpu.sync_copy(xp2_hbm.at[pl.ds(NCH, 1)], idx_v.at[pl.ds(0, 1)])
        pltpu.sync_copy(emb_hbm.at[idx_v.at[0]], rows_v)
        pltpu.sync_copy(rows_v.at[pl.ds(0, 16)], h0_hbm.at[pl.ds(NCH * 128, 16)])


@functools.partial(
    pl.kernel,
    mesh=_MESH,
    out_type=[_f32((2 * N, H))],
    scratch_types=[
        pltpu.VMEM((8, 128), jnp.int32),
        pltpu.VMEM((128, H), jnp.float32),
        pltpu.VMEM((208, H), jnp.float32),
        pltpu.VMEM_SHARED((N, H), jnp.float32),
        pltpu.SemaphoreType.DMA,
    ],
)
def _sc_degree(dst2_hbm, deg_hbm, idx_d, ones_v, zbuf16, dacc, ssem):
    cid = lax.axis_index("c")
    s = lax.axis_index("s")
    zero16 = jnp.zeros((16,), jnp.float32)
    one16 = jnp.ones((16,), jnp.float32)

    @pl.loop(0, 208)
    def _zf(r):
        for k in range(8):
            zbuf16[r, pl.ds(k * 16, 16)] = zero16

    @pl.loop(0, 128)
    def _of(r):
        for k in range(8):
            ones_v[r, pl.ds(k * 16, 16)] = one16

    for k in range(3):
        pltpu.sync_copy(zbuf16, dacc.at[pl.ds(s * 624 + k * 208, 208)])

    @pl.when(s == 0)
    def _ztail():
        pltpu.sync_copy(zbuf16.at[pl.ds(0, 16)], dacc.at[pl.ds(9984, 16)])

    plsc.subcore_barrier()

    @pl.loop(s, 157, step=16)
    def _edges(m):
        def _rows(nrows):
            pltpu.sync_copy(dst2_hbm.at[pl.ds(cid * ERP + m * 8, nrows)],
                            idx_d.at[pl.ds(0, nrows)])
            hs = [pltpu.async_copy(ones_v, dacc.at[idx_d.at[j]], ssem,
                                   add=True)
                  for j in range(nrows)]
            for h in hs:
                h.wait()

        @pl.when(m < 156)
        def _full():
            _rows(8)

        @pl.when(m == 156)
        def _tail():
            _rows(2)

    plsc.subcore_barrier()

    for k in range(3):
        rs = pl.ds(s * 624 + k * 208, 208)
        os = pl.ds(cid * N + s * 624 + k * 208, 208)
        pltpu.sync_copy(dacc.at[rs], deg_hbm.at[os])

    @pl.when(s == 0)
    def _otail():
        pltpu.sync_copy(dacc.at[pl.ds(9984, 16)],
                        deg_hbm.at[pl.ds(cid * N + 9984, 16)])


@functools.partial(
    pl.kernel,
    mesh=_MESH,
    out_type=[_f32((2 * N, H))],
    scratch_types=[
        pltpu.VMEM((8, 128), jnp.int32),
        pltpu.VMEM((8, 128), jnp.int32),
        pltpu.VMEM((8, 128), jnp.int32),
        pltpu.VMEM((128, H), jnp.float32),
        pltpu.VMEM((128, H), jnp.float32),
        pltpu.VMEM((104, H), jnp.float32),
        pltpu.VMEM_SHARED((N, H), jnp.float32),
        pltpu.SemaphoreType.DMA,
        pltpu.SemaphoreType.DMA,
        pltpu.SemaphoreType.DMA,
        pltpu.SemaphoreType.DMA,
    ],
)
def _sc_aggregate(hr_hbm, srcr_hbm, dstr_hbm, agg_hbm,
                  idx_s, idx_d, gidx, buf_a, buf_b, zbuf, acc,
                  sem_a, sem_b, ssem_a, ssem_b):
    cid = lax.axis_index("c")
    s = lax.axis_index("s")
    zero16 = jnp.zeros((16,), jnp.float32)

    @pl.loop(0, 104)
    def _zfill(r):
        for k in range(8):
            zbuf[r, pl.ds(k * 16, 16)] = zero16

    for k in range(6):
        pltpu.sync_copy(zbuf, acc.at[pl.ds(s * 624 + k * 104, 104)])

    @pl.when(s == 0)
    def _ztail():
        pltpu.sync_copy(zbuf.at[pl.ds(0, 16)], acc.at[pl.ds(9984, 16)])

    plsc.subcore_barrier()

    @pl.loop(s, 157, step=16)
    def _edges(m):
        def _rows(nrows):
            pltpu.sync_copy(srcr_hbm.at[pl.ds(m * 8, nrows)],
                            idx_s.at[pl.ds(0, nrows)])
            pltpu.sync_copy(dstr_hbm.at[pl.ds(m * 8, nrows)],
                            idx_d.at[pl.ds(0, nrows)])
            for j in range(nrows):
                for k in range(8):
                    sl = pl.ds(k * 16, 16)
                    gidx[j, sl] = idx_s[j, sl] * 2 + cid
            bufs = (buf_a, buf_b)
            sems = (sem_a, sem_b)
            ssems = (ssem_a, ssem_b)
            pend = [None, None]
            spend = [None, None]
            pend[0] = pltpu.async_copy(hr_hbm.at[gidx.at[0]], bufs[0], sems[0])
            for j in range(nrows):
                pend[j % 2].wait()
                if j + 1 < nrows:
                    if spend[(j + 1) % 2] is not None:
                        spend[(j + 1) % 2].wait()
                    pend[(j + 1) % 2] = pltpu.async_copy(
                        hr_hbm.at[gidx.at[j + 1]], bufs[(j + 1) % 2],
                        sems[(j + 1) % 2])
                spend[j % 2] = pltpu.async_copy(
                    bufs[j % 2], acc.at[idx_d.at[j]], ssems[j % 2], add=True)
            for b in range(2):
                if spend[b] is not None:
                    spend[b].wait()

        @pl.when(m < 156)
        def _full():
            _rows(8)

        @pl.when(m == 156)
        def _tail():
            _rows(2)

    plsc.subcore_barrier()

    for k in range(3):
        rs = pl.ds(s * 624 + k * 208, 208)
        os = pl.ds(cid * N + s * 624 + k * 208, 208)
        pltpu.sync_copy(acc.at[rs], agg_hbm.at[os])

    @pl.when(s == 0)
    def _otail():
        pltpu.sync_copy(acc.at[pl.ds(9984, 16)],
                        agg_hbm.at[pl.ds(cid * N + 9984, 16)])


@functools.partial(
    pl.kernel,
    mesh=_MESH,
    out_type=[_f32((4 * P, H))],
    scratch_types=[
        pltpu.VMEM((8, 128), jnp.int32),
        pltpu.VMEM((8, 128), jnp.int32),
        pltpu.VMEM((1, 128), jnp.int32),
        pltpu.VMEM((1, 128), jnp.int32),
        pltpu.VMEM((128, H), jnp.float32),
        pltpu.VMEM((128, H), jnp.float32),
    ],
)
def _sc_pairs(hr_hbm, pairs_hbm, z_hbm, ia, ib, ga, gb, ra, rb):
    cid = lax.axis_index("c")
    s = lax.axis_index("s")

    side = s // 8
    m = s % 8
    pltpu.sync_copy(pairs_hbm.at[pl.ds(side * 2 * PR + m * 8, 8)], ia)
    pltpu.sync_copy(pairs_hbm.at[pl.ds(side * 2 * PR + PR + m * 8, 8)], ib)
    for j in range(8):
        for k in range(8):
            sl = pl.ds(k * 16, 16)
            ga[0, sl] = ia[j, sl] * 2 + cid
            gb[0, sl] = ib[j, sl] * 2 + cid
        pltpu.sync_copy(hr_hbm.at[ga.at[0]], ra)
        pltpu.sync_copy(hr_hbm.at[gb.at[0]], rb)

        @pl.loop(0, 128)
        def _mul(i):
            for k in range(8):
                sl = pl.ds(k * 16, 16)
                ra[i, sl] = ra[i, sl] * rb[i, sl]

        pltpu.sync_copy(
            ra, z_hbm.at[pl.ds(side * 2 * P + cid * P + (m * 8 + j) * 128, 128)])


def _combine_body(h_ref, alo_ref, ahi_ref, deg_ref, ws_ref, wn_ref, b_ref,
                  o_ref, *, relu):
    inv = 1.0 / jnp.maximum(deg_ref[...][:, 0:1], 1.0)
    acc = jnp.dot(h_ref[...], ws_ref[...], preferred_element_type=jnp.float32)
    acc = acc + jnp.dot(alo_ref[...] * inv, wn_ref[...][:H, :],
                        preferred_element_type=jnp.float32)
    acc = acc + jnp.dot(ahi_ref[...] * inv, wn_ref[...][H:, :],
                        preferred_element_type=jnp.float32)
    acc = acc + b_ref[...]
    if relu:
        acc = jnp.maximum(acc, 0.0)
    o_ref[...] = acc


def _combine(h, agg_lo, agg_hi, deg, W_self, W_neigh, b, relu):
    return pl.pallas_call(
        functools.partial(_combine_body, relu=relu),
        grid=(N // BN,),
        in_specs=[
            pl.BlockSpec((BN, D), lambda i: (i, 0)),
            pl.BlockSpec((BN, H), lambda i: (i, 0)),
            pl.BlockSpec((BN, H), lambda i: (i, 0)),
            pl.BlockSpec((BN, H), lambda i: (i, 0)),
            pl.BlockSpec((D, D), lambda i: (0, 0)),
            pl.BlockSpec((D, D), lambda i: (0, 0)),
            pl.BlockSpec((1, D), lambda i: (0, 0)),
        ],
        out_specs=pl.BlockSpec((BN, D), lambda i: (i, 0)),
        out_shape=_f32((N, D)),
    )(h, agg_lo, agg_hi, deg, W_self, W_neigh, b.reshape(1, D))


def _decoder_body(zpl_ref, zph_ref, znl_ref, znh_ref, w1_ref, b1_ref, w2_ref,
                  b2_ref, w3_ref, b3_ref, op_ref, on_ref):
    for zl_ref, zh_ref, o_ref in ((zpl_ref, zph_ref, op_ref),
                                  (znl_ref, znh_ref, on_ref)):
        z = jnp.dot(zl_ref[...], w1_ref[...][:H, :],
                    preferred_element_type=jnp.float32)
        z = z + jnp.dot(zh_ref[...], w1_ref[...][H:, :],
                        preferred_element_type=jnp.float32)
        z = jnp.maximum(z + b1_ref[...], 0.0)
        z = jnp.dot(z, w2_ref[...], preferred_element_type=jnp.float32)
        z = jnp.maximum(z + b2_ref[...], 0.0)
        o_ref[...] = jnp.sum(z * w3_ref[...], axis=1, keepdims=True) + b3_ref[...]


def _decoder(zp_lo, zp_hi, zn_lo, zn_hi, dW1, db1, dW2, db2, dW3, db3):
    return pl.pallas_call(
        _decoder_body,
        grid=(P // BP,),
        in_specs=[
            pl.BlockSpec((BP, H), lambda i: (i, 0)),
            pl.BlockSpec((BP, H), lambda i: (i, 0)),
            pl.BlockSpec((BP, H), lambda i: (i, 0)),
            pl.BlockSpec((BP, H), lambda i: (i, 0)),
            pl.BlockSpec((D, D), lambda i: (0, 0)),
            pl.BlockSpec((1, D), lambda i: (0, 0)),
            pl.BlockSpec((D, D), lambda i: (0, 0)),
            pl.BlockSpec((1, D), lambda i: (0, 0)),
            pl.BlockSpec((1, D), lambda i: (0, 0)),
            pl.BlockSpec((1, 1), lambda i: (0, 0)),
        ],
        out_specs=[
            pl.BlockSpec((BP, 1), lambda i: (i, 0)),
            pl.BlockSpec((BP, 1), lambda i: (i, 0)),
        ],
        out_shape=[_f32((P, 1)), _f32((P, 1))],
    )(zp_lo, zp_hi, zn_lo, zn_hi, dW1, db1.reshape(1, D), dW2,
      db2.reshape(1, D), dW3.reshape(1, D), db3.reshape(1, 1))


def kernel(x, edge_index1, edge_index2, pos_src, pos_dst, neg_src, neg_dst,
           emb, W_self1, W_neigh1, b1, W_self2, W_neigh2, b2,
           dW1, db1, dW2, db2, dW3, db3):
    i32 = jnp.int32
    x = x.astype(i32)
    xp2 = jnp.concatenate([x, jnp.zeros((10240 - N,), i32)]).reshape(80, 128)
    src1r = edge_index1[0].astype(i32).reshape(ER, 128)
    dst1r = edge_index1[1].astype(i32).reshape(ER, 128)
    src2r = edge_index2[0].astype(i32).reshape(ER, 128)
    dst2r = edge_index2[1].astype(i32).reshape(ER, 128)
    zpad = jnp.zeros((ERP - ER, 128), i32)
    dst_both = jnp.concatenate([dst1r, zpad, dst2r, zpad], axis=0)
    pairs = jnp.concatenate(
        [pos_src.astype(i32).reshape(PR, 128),
         pos_dst.astype(i32).reshape(PR, 128),
         neg_src.astype(i32).reshape(PR, 128),
         neg_dst.astype(i32).reshape(PR, 128)], axis=0)

    (h0,) = _sc_prepare(xp2, emb)
    (deg_both,) = _sc_degree(dst_both)
    (agg1,) = _sc_aggregate(h0.reshape(2 * N, H), src1r, dst1r)
    h1 = _combine(h0, agg1[:N], agg1[N:], deg_both[:N],
                  W_self1, W_neigh1, b1, relu=True)

    (agg2,) = _sc_aggregate(h1.reshape(2 * N, H), src2r, dst2r)
    h2 = _combine(h1, agg2[:N], agg2[N:], deg_both[N:],
                  W_self2, W_neigh2, b2, relu=False)

    (z,) = _sc_pairs(h2.reshape(2 * N, H), pairs)
    return tuple(_decoder(z[0:P], z[P:2 * P], z[2 * P:3 * P], z[3 * P:],
                          dW1, db1, dW2, db2, dW3, db3))
